# Initial kernel scaffold; baseline (speedup 1.0000x reference)
import jax
import jax.numpy as jnp
from jax.experimental import pallas as pl


def kernel(x, edge_index, batch, W, a_src, a_dst, bias, lin_w, lin_b):
    raise NotImplementedError("write your pallas kernel here")


# speedup vs baseline: 7.0110x; 7.0110x over previous
import functools

import jax
import jax.numpy as jnp
from jax.experimental import pallas as pl

N = 10000
E = 320000
D_IN = 128
HID = 256
HEADS = 2
NHOP = 5
NCLS = 10
NGRAPH = 64

_ROWS = 1000


def _proj_body(x_ref, W_ref, a_src_ref, a_dst_ref, h_ref, coef_ref):
    h = jnp.dot(x_ref[...], W_ref[...], preferred_element_type=jnp.float32)
    h_ref[...] = h
    hh = h.reshape(_ROWS, HEADS, HID)
    als = jnp.sum(hh * a_src_ref[...][None], axis=-1)
    ald = jnp.sum(hh * a_dst_ref[...][None], axis=-1)
    coef_ref[...] = jnp.concatenate([als, ald], axis=-1)


def _project(x, W, a_src, a_dst):
    grid = N // _ROWS
    return pl.pallas_call(
        _proj_body,
        grid=(grid,),
        in_specs=[
            pl.BlockSpec((_ROWS, D_IN), lambda i: (i, 0)),
            pl.BlockSpec((D_IN, HEADS * HID), lambda i: (0, 0)),
            pl.BlockSpec((HEADS, HID), lambda i: (0, 0)),
            pl.BlockSpec((HEADS, HID), lambda i: (0, 0)),
        ],
        out_specs=[
            pl.BlockSpec((_ROWS, HEADS * HID), lambda i: (i, 0)),
            pl.BlockSpec((_ROWS, 4), lambda i: (i, 0)),
        ],
        out_shape=[
            jax.ShapeDtypeStruct((N, HEADS * HID), jnp.float32),
            jax.ShapeDtypeStruct((N, 4), jnp.float32),
        ],
    )(x, W, a_src, a_dst)


def _final_body(u_ref, h_ref, inv_ref, lw_ref, b_ref, bias_ref, o_ref):
    acc = jnp.zeros((NGRAPH, NCLS), jnp.float32)
    for hd in range(HEADS):
        u = u_ref[:, hd * NGRAPH:(hd + 1) * NGRAPH]
        hmat = h_ref[:, hd * HID:(hd + 1) * HID]
        pooled = jax.lax.dot_general(
            u, hmat, (((0,), (0,)), ((), ())),
            preferred_element_type=jnp.float32)
        acc = acc + jnp.dot(pooled, lw_ref[...],
                            preferred_element_type=jnp.float32)
    acc = acc * inv_ref[...] / HEADS
    bias_term = jnp.dot(bias_ref[...].reshape(1, HID), lw_ref[...],
                        preferred_element_type=jnp.float32)
    o_ref[...] = acc + bias_term + b_ref[...][None, :]


def _finalize(u, h, inv_cnt, lin_w, lin_b, bias):
    return pl.pallas_call(
        _final_body,
        out_shape=jax.ShapeDtypeStruct((NGRAPH, NCLS), jnp.float32),
    )(u, h, inv_cnt, lin_w, lin_b, bias)


def kernel(x, edge_index, batch, W, a_src, a_dst, bias, lin_w, lin_b):
    src, dst = edge_index[0], edge_index[1]
    h, coef = _project(x, W, a_src, a_dst)

    e = jax.nn.leaky_relu(coef[src, :HEADS] + coef[dst, HEADS:], 0.2)
    ex = jnp.exp(e)
    denom = jax.ops.segment_sum(ex, dst, num_segments=N)
    alpha = ex / jnp.maximum(denom[dst], 1e-16)

    u0 = (batch[:, None] == jnp.arange(NGRAPH)[None, :]).astype(jnp.float32)
    cnt = u0.sum(0)
    inv_cnt = (1.0 / jnp.maximum(cnt, 1.0))[:, None]

    us = []
    for hd in range(HEADS):
        u = u0
        a = alpha[:, hd:hd + 1]
        for _ in range(NHOP):
            u = jax.ops.segment_sum(a * u[dst], src, num_segments=N)
        us.append(u)
    u = jnp.concatenate(us, axis=1)

    return _finalize(u, h, inv_cnt, lin_w, lin_b, bias)


# speedup vs baseline: 101.4489x; 14.4700x over previous
import functools
import math

import jax
import jax.numpy as jnp
from jax import lax
from jax.experimental import pallas as pl
from jax.experimental.pallas import tpu as pltpu
from jax.experimental.pallas import tpu_sc as plsc

N = 10000
E = 320000
D_IN = 128
HID = 256
HEADS = 2
NHOP = 5
NCLS = 10
NGRAPH = 64

NTILE = 16
EPT = E // NTILE
CH = 128
NCH = 158
NCH2 = NCH // 2
EPAD = NCH * CH
NPAD = 10240
RPT = NPAD // NTILE

_ROWS = 1000


def _proj_body(x_ref, W_ref, a_src_ref, a_dst_ref, h_ref, coef_ref):
    h = jnp.dot(x_ref[...], W_ref[...], preferred_element_type=jnp.float32)
    h_ref[...] = h
    hh = h.reshape(_ROWS, HEADS, HID)
    als = jnp.sum(hh * a_src_ref[...][None], axis=-1)
    ald = jnp.sum(hh * a_dst_ref[...][None], axis=-1)
    coef_ref[...] = jnp.concatenate([als, ald], axis=-1)


def _project(x, W, a_src, a_dst):
    grid = N // _ROWS
    return pl.pallas_call(
        _proj_body,
        grid=(grid,),
        in_specs=[
            pl.BlockSpec((_ROWS, D_IN), lambda i: (i, 0)),
            pl.BlockSpec((D_IN, HEADS * HID), lambda i: (0, 0)),
            pl.BlockSpec((HEADS, HID), lambda i: (0, 0)),
            pl.BlockSpec((HEADS, HID), lambda i: (0, 0)),
        ],
        out_specs=[
            pl.BlockSpec((_ROWS, HEADS * HID), lambda i: (i, 0)),
            pl.BlockSpec((_ROWS, 4), lambda i: (i, 0)),
        ],
        out_shape=[
            jax.ShapeDtypeStruct((N, HEADS * HID), jnp.float32),
            jax.ShapeDtypeStruct((N, 4), jnp.float32),
        ],
    )(x, W, a_src, a_dst)


def _final_body(u_ref, h_ref, inv_ref, lw_ref, b_ref, bias_ref, o_ref):
    acc = jnp.zeros((NGRAPH, NCLS), jnp.float32)
    for hd in range(HEADS):
        u = u_ref[:, hd * NGRAPH:(hd + 1) * NGRAPH]
        hmat = h_ref[:, hd * HID:(hd + 1) * HID]
        pooled = jax.lax.dot_general(
            u, hmat, (((0,), (0,)), ((), ())),
            preferred_element_type=jnp.float32)
        acc = acc + jnp.dot(pooled, lw_ref[...],
                            preferred_element_type=jnp.float32)
    acc = acc * inv_ref[...] / HEADS
    bias_term = jnp.dot(bias_ref[...].reshape(1, HID), lw_ref[...],
                        preferred_element_type=jnp.float32)
    o_ref[...] = acc + bias_term + b_ref[...][None, :]


def _finalize(u, h, inv_cnt, lin_w, lin_b, bias):
    return pl.pallas_call(
        _final_body,
        out_shape=jax.ShapeDtypeStruct((NGRAPH, NCLS), jnp.float32),
    )(u, h, inv_cnt, lin_w, lin_b, bias)


def _sc_body(src_hbm, dst_hbm, coef_hbm, batch_hbm, u_out_hbm,
             esrc, edst, alpha_t, as_t, ad_t, dt, bt, rb, zbuf,
             u_acc, denom_sp, sem):
    c = lax.axis_index("c")
    s = lax.axis_index("s")
    iota = lax.iota(jnp.int32, 16)

    pltpu.sync_copy(coef_hbm.at[c, 0], as_t)
    pltpu.sync_copy(coef_hbm.at[c, 1], ad_t)
    pltpu.sync_copy(batch_hbm.at[pl.ds(s * RPT, RPT)], bt)

    def zb(i, _):
        zbuf[pl.ds(i * 16, 16)] = jnp.zeros((16,), jnp.float32)
        return 0
    lax.fori_loop(0, RPT // 16, zb, 0)

    pltpu.sync_copy(zbuf.at[pl.ds(0, RPT)], denom_sp.at[pl.ds(s * RPT, RPT)])

    def zr(i, _):
        for f in range(NGRAPH // 16):
            rb[1, i, pl.ds(f * 16, 16)] = jnp.zeros((16,), jnp.float32)
        return 0
    lax.fori_loop(0, CH, zr, 0)

    def u0_block2(k, _):
        def u0_row16(rr, _):
            bv = bt[pl.ds(k * CH + rr * 16, 16)]
            for i in range(16):
                g = bv[i]
                for f in range(NGRAPH // 16):
                    rb[0, rr * 16 + i, pl.ds(f * 16, 16)] = jnp.where(
                        iota + (f * 16) == g, 1.0, 0.0)
            return 0
        lax.fori_loop(0, CH // 16, u0_row16, 0)
        row0 = s * RPT + k * CH
        pltpu.sync_copy(rb.at[0], u_out_hbm.at[c, pl.ds(row0, CH), :])
        pltpu.sync_copy(rb.at[1], u_acc.at[pl.ds(row0, CH), :])
        return 0
    lax.fori_loop(0, RPT // CH, u0_block2, 0)

    plsc.subcore_barrier()

    for half in range(2):
        pltpu.sync_copy(src_hbm.at[s, pl.ds(half * NCH2, NCH2)], esrc)
        pltpu.sync_copy(dst_hbm.at[s, pl.ds(half * NCH2, NCH2)], edst)

        def att_chunk(k, _):
            ci = half * NCH2 + k
            base = ci * CH
            for jj in range(CH // 16):
                ev_idx = base + jj * 16 + iota
                srcv = esrc[k, 0, pl.ds(jj * 16, 16)]
                dstv = edst[k, 0, pl.ds(jj * 16, 16)]
                av = plsc.load_gather(as_t, [srcv])
                dv = plsc.load_gather(ad_t, [dstv])
                e = av + dv
                e = jnp.where(e < 0.0, e * jnp.float32(0.2), e)
                ex = jnp.exp(e)
                ex = jnp.where(ev_idx < EPT, ex, 0.0)
                alpha_t[ci, 0, pl.ds(jj * 16, 16)] = ex
            pltpu.sync_copy(alpha_t.at[ci, 0], denom_sp.at[edst.at[k, 0]],
                            add=True)
            return 0
        lax.fori_loop(0, NCH2, att_chunk, 0)

    plsc.subcore_barrier()

    pltpu.sync_copy(denom_sp.at[pl.ds(0, N)], dt)
    for half in range(2):
        pltpu.sync_copy(dst_hbm.at[s, pl.ds(half * NCH2, NCH2)], edst)

        def div_chunk(k, _):
            ci = half * NCH2 + k
            for jj in range(CH // 16):
                dstv = edst[k, 0, pl.ds(jj * 16, 16)]
                dv = plsc.load_gather(dt, [dstv])
                ex = alpha_t[ci, 0, pl.ds(jj * 16, 16)]
                alpha_t[ci, 0, pl.ds(jj * 16, 16)] = ex / jnp.maximum(
                    dv, jnp.float32(1e-16))
            return 0
        lax.fori_loop(0, NCH2, div_chunk, 0)

    plsc.subcore_barrier()

    for hop in range(NHOP):
        for half in range(2):
            pltpu.sync_copy(src_hbm.at[s, pl.ds(half * NCH2, NCH2)], esrc)
            pltpu.sync_copy(dst_hbm.at[s, pl.ds(half * NCH2, NCH2)], edst)

            def hop_chunk(k, _):
                ci = half * NCH2 + k
                pltpu.async_copy(
                    u_out_hbm.at[c].at[edst.at[k, 0]], rb.at[0], sem).wait()
                for jj in range(CH // 16):
                    av = alpha_t[ci, 0, pl.ds(jj * 16, 16)]
                    for i in range(16):
                        r = jj * 16 + i
                        a_i = av[i]
                        for f in range(NGRAPH // 16):
                            rb[0, r, pl.ds(f * 16, 16)] = (
                                rb[0, r, pl.ds(f * 16, 16)] * a_i)
                pltpu.sync_copy(rb.at[0], u_acc.at[esrc.at[k, 0]], add=True)
                return 0
            lax.fori_loop(0, NCH2, hop_chunk, 0)

        plsc.subcore_barrier()

        def flush_blk(k, _):
            row0 = s * RPT + k * CH
            pltpu.sync_copy(u_acc.at[pl.ds(row0, CH), :], rb.at[0])
            pltpu.sync_copy(rb.at[0], u_out_hbm.at[c, pl.ds(row0, CH), :])
            pltpu.sync_copy(rb.at[1], u_acc.at[pl.ds(row0, CH), :])
            return 0
        lax.fori_loop(0, RPT // CH, flush_blk, 0)
        plsc.subcore_barrier()


def _sc_propagate(srcp, dstp, coefs, batchp):
    mesh = plsc.VectorSubcoreMesh(core_axis_name="c", subcore_axis_name="s")
    f = pl.kernel(
        _sc_body,
        mesh=mesh,
        compiler_params=pltpu.CompilerParams(
            needs_layout_passes=False, use_tc_tiling_on_sc=False),
        out_type=jax.ShapeDtypeStruct((HEADS, NPAD, NGRAPH), jnp.float32),
        scratch_types=[
            pltpu.VMEM((NCH2, 1, CH), jnp.int32),
            pltpu.VMEM((NCH2, 1, CH), jnp.int32),
            pltpu.VMEM((NCH, 1, CH), jnp.float32),
            pltpu.VMEM((N,), jnp.float32),
            pltpu.VMEM((N,), jnp.float32),
            pltpu.VMEM((N,), jnp.float32),
            pltpu.VMEM((RPT,), jnp.int32),
            pltpu.VMEM((2, CH, NGRAPH), jnp.float32),
            pltpu.VMEM((RPT,), jnp.float32),
            pltpu.VMEM_SHARED((NPAD, NGRAPH), jnp.float32),
            pltpu.VMEM_SHARED((NPAD,), jnp.float32),
            pltpu.SemaphoreType.DMA,
        ],
    )
    return f(srcp, dstp, coefs, batchp)


def kernel(x, edge_index, batch, W, a_src, a_dst, bias, lin_w, lin_b):
    src, dst = edge_index[0], edge_index[1]
    h, coef = _project(x, W, a_src, a_dst)

    pad = jnp.zeros((NTILE, EPAD - EPT), jnp.int32)
    srcp = jnp.concatenate([src.reshape(NTILE, EPT), pad], axis=1)
    srcp = srcp.reshape(NTILE, NCH, 1, CH)
    dstp = jnp.concatenate([dst.reshape(NTILE, EPT), pad], axis=1)
    dstp = dstp.reshape(NTILE, NCH, 1, CH)
    coefs = jnp.stack([
        jnp.stack([coef[:, 0], coef[:, 2]]),
        jnp.stack([coef[:, 1], coef[:, 3]]),
    ])
    batchp = jnp.concatenate([batch, jnp.zeros((NPAD - N,), jnp.int32)])

    u5 = _sc_propagate(srcp, dstp, coefs, batchp)

    u = jnp.concatenate([u5[0, :N], u5[1, :N]], axis=1)

    u0 = (batch[:, None] == jnp.arange(NGRAPH)[None, :]).astype(jnp.float32)
    cnt = u0.sum(0)
    inv_cnt = (1.0 / jnp.maximum(cnt, 1.0))[:, None]

    return _finalize(u, h, inv_cnt, lin_w, lin_b, bias)
